# baseline (speedup 1.0000x reference)
import jax
import jax.numpy as jnp
from jax.experimental import pallas as pl

N = 1024
B = 4
H = 32
T_ENC = 12
T_DEC = 12
NM = 3

F32 = jnp.float32
BF16 = jnp.bfloat16


def _dot(a, b):
    return jnp.dot(a, b, preferred_element_type=F32)


def _fused_kernel(adj_ref, xenc_ref, wall_ref, ball_ref,
                  wp_ref, bp_ref, out_ref):
    adj = adj_ref[...]

    def gconv(p0, widx, width):
        p1 = _dot(adj, p0.astype(BF16))
        p2 = 2.0 * _dot(adj, p1.astype(BF16)) - p0
        out = ball_ref[widx, :, :width]
        for m, pm in enumerate((p0, p1, p2)):
            out = out + _dot(pm, wall_ref[widx, m, :, :width])
        return out

    def cell(x, h, k):
        p0 = jnp.concatenate([h, x], axis=1)
        v = 0.5 * (1.0 + jnp.tanh(0.5 * gconv(p0, k, 2 * B * H)))
        r = v[:, :B * H]
        u = v[:, B * H:]
        c0 = jnp.concatenate([r * h, x], axis=1)
        c = jnp.tanh(gconv(c0, 2 + k, B * H))
        return u * h + (1.0 - u) * c

    h = jnp.zeros((N, B * H), F32)
    for t in range(T_ENC):
        h = cell(xenc_ref[t].T, h, 0)

    dec = jnp.zeros((N, B), F32)
    for t in range(T_DEC):
        h = cell(dec, h, 1)
        dec = _dot(h, wp_ref[...]) + bp_ref[...]
        out_ref[t] = dec.T


def kernel(inputs, targets, adj_matrix, W_eg, b_eg, W_ec, b_ec,
           W_dg, b_dg, W_dc, b_dc, W_pred, b_pred):
    del targets
    xenc = inputs.reshape(T_ENC, B, N)

    eye = jnp.eye(B, dtype=F32)
    zpad = jnp.zeros((2, (1 + H) * NM, H), F32)
    Wc4 = jnp.concatenate([jnp.stack([W_ec, W_dc]), zpad], axis=2)
    W4 = jnp.concatenate([jnp.stack([W_eg, W_dg]), Wc4], axis=0)
    Wr = W4.reshape(4, 1 + H, NM, 2, H)
    S = jnp.einsum('khmgj,bc->kmbhgcj', Wr[:, 1:], eye)
    S = S.reshape(4, NM, B * H, 2 * B * H)
    X = jnp.einsum('kmgj,bc->kmbgcj', Wr[:, 0], eye)
    X = X.reshape(4, NM, B, 2 * B * H)
    wall = jnp.concatenate([S, X], axis=2)

    bc4 = jnp.concatenate([jnp.stack([b_ec, b_dc]),
                           jnp.zeros((2, H), F32)], axis=1)
    b4 = jnp.concatenate([jnp.stack([b_eg, b_dg]), bc4], axis=0)
    ball = jnp.tile(b4.reshape(4, 2, 1, H), (1, 1, B, 1))
    ball = ball.reshape(4, 1, 2 * B * H)

    wp = jnp.einsum('j,bc->bjc', W_pred[:, 0], eye).reshape(B * H, B)
    bp = jnp.broadcast_to(b_pred.reshape(1, 1), (1, B))

    out = pl.pallas_call(
        _fused_kernel,
        out_shape=jax.ShapeDtypeStruct((T_DEC, B, N), F32),
    )(adj_matrix.astype(BF16), xenc, wall, ball, wp, bp)

    return out


# speedup vs baseline: 1.0535x; 1.0535x over previous
import jax
import jax.numpy as jnp
from jax.experimental import pallas as pl

N = 1024
B = 4
H = 32
T_ENC = 12
T_DEC = 12
NM = 3

F32 = jnp.float32
BF16 = jnp.bfloat16


def _dot(a, b):
    return jnp.dot(a, b, preferred_element_type=F32)


def _fused_kernel(adj_ref, xenc_ref, wg_ref, bg_ref, wc_ref, bc_ref,
                  wp_ref, bp_ref, out_ref):
    adj = adj_ref[...]

    def gconv(p0, k, w_ref, b_ref):
        p1 = _dot(adj, p0.astype(BF16))
        p2 = 2.0 * _dot(adj, p1.astype(BF16)) - p0
        return (b_ref[k] + _dot(p0, w_ref[k, 0]) + _dot(p1, w_ref[k, 1])
                + _dot(p2, w_ref[k, 2]))

    def cell(x, h, k):
        p0 = jnp.concatenate([h, x], axis=1)
        v = 0.5 * (1.0 + jnp.tanh(0.5 * gconv(p0, k, wg_ref, bg_ref)))
        r = v[:, :B * H]
        u = v[:, B * H:]
        c0 = jnp.concatenate([r * h, x], axis=1)
        c = jnp.tanh(gconv(c0, k, wc_ref, bc_ref))
        return u * h + (1.0 - u) * c

    h = jnp.zeros((N, B * H), F32)
    for t in range(T_ENC):
        h = cell(xenc_ref[t].T, h, 0)

    dec = jnp.zeros((N, B), F32)
    for t in range(T_DEC):
        h = cell(dec, h, 1)
        dec = _dot(h, wp_ref[...]) + bp_ref[...]
        out_ref[t] = dec.T


def _pack(W2, bias2, D, eye):
    if D == 2 * H:
        Wr = W2.reshape(2, 1 + H, NM, 2, H)
        S = jnp.einsum('khmgj,bc->kmbhgcj', Wr[:, 1:], eye)
        S = S.reshape(2, NM, B * H, 2 * B * H)
        X = jnp.einsum('kmgj,bc->kmbgcj', Wr[:, 0], eye)
        X = X.reshape(2, NM, B, 2 * B * H)
        bp = jnp.tile(bias2.reshape(2, 2, 1, H), (1, 1, B, 1))
        bp = bp.reshape(2, 1, 2 * B * H)
    else:
        Wr = W2.reshape(2, 1 + H, NM, H)
        S = jnp.einsum('khmj,bc->kmbhcj', Wr[:, 1:], eye)
        S = S.reshape(2, NM, B * H, B * H)
        X = jnp.einsum('kmj,bc->kmbcj', Wr[:, 0], eye)
        X = X.reshape(2, NM, B, B * H)
        bp = jnp.tile(bias2.reshape(2, 1, H), (1, B, 1)).reshape(2, 1, B * H)
    W = jnp.concatenate([S, X], axis=2)
    return W, bp


def kernel(inputs, targets, adj_matrix, W_eg, b_eg, W_ec, b_ec,
           W_dg, b_dg, W_dc, b_dc, W_pred, b_pred):
    del targets
    xenc = inputs.reshape(T_ENC, B, N)

    eye = jnp.eye(B, dtype=F32)
    wg, bg = _pack(jnp.stack([W_eg, W_dg]), jnp.stack([b_eg, b_dg]),
                   2 * H, eye)
    wc, bc = _pack(jnp.stack([W_ec, W_dc]), jnp.stack([b_ec, b_dc]), H, eye)

    wp = jnp.einsum('j,bc->bjc', W_pred[:, 0], eye).reshape(B * H, B)
    bp = jnp.broadcast_to(b_pred.reshape(1, 1), (1, B))

    out = pl.pallas_call(
        _fused_kernel,
        out_shape=jax.ShapeDtypeStruct((T_DEC, B, N), F32),
    )(adj_matrix.astype(BF16), xenc, wg, bg, wc, bc, wp, bp)

    return out
